# baseline (speedup 1.0000x reference)
import functools

import jax
import jax.numpy as jnp
from jax import lax
from jax.experimental import pallas as pl
from jax.experimental.pallas import tpu as pltpu
from jax.experimental.pallas import tpu_sc as plsc

D = 32
L = 50
B = 16384
N = B * L
NC = 2
NS = 16
NW = NC * NS
CH1 = 1024
NCH1 = N // NW // CH1
CH2 = 1024
UNITS = L * (B // CH2)
UPW = UNITS // NW

_mesh = plsc.VectorSubcoreMesh(core_axis_name="c", subcore_axis_name="s")


@jax.jit
def _sc_pipeline(xt_flat, wr, wi):
    @functools.partial(
        pl.kernel,
        mesh=_mesh,
        out_type=[
            jax.ShapeDtypeStruct((N, D), jnp.float32),
            jax.ShapeDtypeStruct((N, D), jnp.float32),
        ],
        scratch_types=[
            pltpu.VMEM((CH1,), jnp.int32),
            pltpu.VMEM((CH1, D), jnp.float32),
            pltpu.VMEM((CH1, D), jnp.float32),
            pltpu.SemaphoreType.DMA,
            pltpu.SemaphoreType.DMA,
        ],
        compiler_params=pltpu.CompilerParams(use_tc_tiling_on_sc=False),
    )
    def k1(x_hbm, wr_hbm, wi_hbm, outr_hbm, outi_hbm, idx_v, rr_v, ri_v, sem_r, sem_i):
        wid = lax.axis_index("s") * NC + lax.axis_index("c")
        base = wid * (N // NW)

        def body(ci, _):
            off = base + ci * CH1
            pltpu.sync_copy(x_hbm.at[pl.ds(off, CH1)], idx_v)
            cp_r = pltpu.async_copy(wr_hbm.at[idx_v], rr_v, sem_r)
            cp_i = pltpu.async_copy(wi_hbm.at[idx_v], ri_v, sem_i)
            cp_r.wait()
            cp_i.wait()
            pltpu.sync_copy(rr_v, outr_hbm.at[pl.ds(off, CH1)])
            pltpu.sync_copy(ri_v, outi_hbm.at[pl.ds(off, CH1)])
            return ()

        lax.fori_loop(0, NCH1, body, (), unroll=False)

    r_lin, i_lin = k1(xt_flat, wr, wi)
    r_flat = r_lin.reshape(-1)
    i_flat = i_lin.reshape(-1)

    @functools.partial(
        pl.kernel,
        mesh=_mesh,
        out_type=[
            jax.ShapeDtypeStruct((L, D, B), jnp.float32),
            jax.ShapeDtypeStruct((L, D, B), jnp.float32),
        ],
        scratch_types=[
            pltpu.VMEM((CH2 * D,), jnp.float32),
            pltpu.VMEM((D, CH2), jnp.float32),
        ],
        compiler_params=pltpu.CompilerParams(use_tc_tiling_on_sc=True, needs_layout_passes=False),
    )
    def k2(rf_hbm, if_hbm, outr_hbm, outi_hbm, rows_v, t_v):
        wid = lax.axis_index("s") * NC + lax.axis_index("c")
        iota32 = lax.iota(jnp.int32, 16) * D

        def transpose_plane(src_hbm, dst_hbm, l, b0):
            base = (l * B + b0) * D
            pltpu.sync_copy(src_hbm.at[pl.ds(base, CH2 * D)], rows_v)

            def grp_body(g, _):
                colbase = iota32 + g * (16 * D)
                for d in range(D):
                    vals = plsc.load_gather(rows_v, [colbase + d])
                    t_v[d, pl.ds(g * 16, 16)] = vals
                return ()

            lax.fori_loop(0, CH2 // 16, grp_body, (), unroll=False)
            pltpu.sync_copy(t_v, dst_hbm.at[l, :, pl.ds(b0, CH2)])

        def body(ui, _):
            u = wid * UPW + ui
            l = u // (B // CH2)
            b0 = (u % (B // CH2)) * CH2
            transpose_plane(rf_hbm, outr_hbm, l, b0)
            transpose_plane(if_hbm, outi_hbm, l, b0)
            return ()

        lax.fori_loop(0, UPW, body, (), unroll=False)

    return k2(r_flat, i_flat)


def kernel(x, W_real, W_imag):
    xt_flat = x.T.reshape(-1)
    r_t, i_t = _sc_pipeline(xt_flat, W_real, W_imag)
    return lax.complex(r_t, i_t).transpose(2, 0, 1)
